# baseline (speedup 1.0000x reference)
import jax
import jax.numpy as jnp
from jax.experimental import pallas as pl
from jax.experimental.pallas import tpu as pltpu
from jax.experimental.pallas import tpu_sc as plsc
import functools
from jax import lax

ROT = 32
HALF = ROT // 2
BASE = 10000.0


def _router_body(x_ref, selcat_ref, logits_ref):
    logits_ref[...] = jax.lax.dot_general(
        x_ref[...].astype(jnp.bfloat16), selcat_ref[...].astype(jnp.bfloat16),
        (((1,), (0,)), ((), ())),
        preferred_element_type=jnp.float32)


def _make_sc_top2(S, Z):
    info = plsc.get_sparse_core_info()
    NC, NS = info.num_cores, info.num_subcores
    NW = NC * NS
    CB = S // (NW // 2)
    ZH = Z // 2
    mesh = plsc.VectorSubcoreMesh(core_axis_name="c", subcore_axis_name="s")

    @functools.partial(
        pl.kernel,
        out_type=jax.ShapeDtypeStruct((Z, S), jnp.float32),
        mesh=mesh,
        scratch_types=[
            pltpu.VMEM((ZH, CB), jnp.float32),
            pltpu.VMEM((ZH, CB), jnp.float32),
        ],
    )
    def sc_top2(logitsT_hbm, out_hbm, lv, ov):
        wid = lax.axis_index("s") * NC + lax.axis_index("c")
        p = wid // 2
        q = wid % 2
        t0 = p * CB
        r0 = q * ZH
        pltpu.sync_copy(logitsT_hbm.at[pl.ds(r0, ZH), pl.ds(t0, CB)], lv)
        for g in range(ZH // 8):
            for tb in range(CB // 16):
                cols = pl.ds(tb * 16, 16)
                ls = [lv[g * 8 + e, cols] for e in range(8)]
                m1 = ls[0]
                a1 = jnp.zeros((16,), jnp.int32)
                m2 = jnp.full((16,), -1e30, jnp.float32)
                a2 = jnp.full((16,), 8, jnp.int32)
                for e in range(1, 8):
                    gt1 = ls[e] > m1
                    gt2 = ls[e] > m2
                    a2 = jnp.where(gt1, a1, jnp.where(gt2, e, a2))
                    m2 = jnp.where(gt1, m1, jnp.where(gt2, ls[e], m2))
                    a1 = jnp.where(gt1, e, a1)
                    m1 = jnp.where(gt1, ls[e], m1)
                for e in range(8):
                    sel = (a1 == e) | (a2 == e)
                    ov[g * 8 + e, cols] = jnp.where(
                        sel, 1.0 / (1.0 + jnp.exp(-ls[e])), 0.0)
        pltpu.sync_copy(ov, out_hbm.at[pl.ds(r0, ZH), pl.ds(t0, CB)])

    return sc_top2


def _mega_body(x_ref, cosT_ref, sinT_ref, wq_ref, wk_ref, wv_ref, wo_ref,
               wselT_ref, woutT_ref, g1_ref, bb1_ref,
               w1_ref, b1_ref, w2_ref, b2_ref, g2_ref, bb2_ref, out_ref,
               xh_s, wcat_s, wos_s, q_s, k_s, v_s, acc_s, xn_s, *, nheads):
    h = pl.program_id(0)
    nh = pl.num_programs(0)
    S, D = x_ref.shape
    E = wq_ref.shape[1]
    P = wq_ref.shape[3]
    G = 3 * P
    SB = min(256, S)
    nsb = S // SB
    scale = P ** -0.5

    @pl.when(h == 0)
    def _init():
        xh_s[...] = x_ref[...].astype(jnp.bfloat16)

    def _do_head():
        for e in range(E):
            wcat_s[:, e * G:e * G + P] = wq_ref[0, e].astype(jnp.bfloat16)
            wcat_s[:, e * G + P:e * G + 2 * P] = wk_ref[0, e].astype(jnp.bfloat16)
            wcat_s[:, e * G + 2 * P:(e + 1) * G] = wv_ref[0, e].astype(jnp.bfloat16)
        wos_s[...] = jnp.concatenate(
            [wo_ref[0, e] for e in range(E)], axis=0).astype(jnp.bfloat16)

        cosT = cosT_ref[...]
        sinT = sinT_ref[...]

        def rope_t(tt, sb):
            c = cosT[:, sb * SB:(sb + 1) * SB]
            s = sinT[:, sb * SB:(sb + 1) * SB]
            t1 = tt[0:HALF, :]
            t2 = tt[HALF:ROT, :]
            return jnp.concatenate(
                [t1 * c - t2 * s, t1 * s + t2 * c, tt[ROT:, :]], axis=0)

        for sb in range(nsb):
            rows = pl.ds(sb * SB, SB)
            xb = xh_s[rows, :]
            qkv = jax.lax.dot_general(
                xb, wcat_s[...], (((1,), (0,)), ((), ())),
                preferred_element_type=jnp.float32).astype(jnp.bfloat16)
            w = wselT_ref[:, rows].T.astype(jnp.bfloat16)
            acc = qkv[:, 0:G] * w[:, 0:1]
            for e in range(1, E):
                acc = acc + qkv[:, e * G:(e + 1) * G] * w[:, e:e + 1]
            q, k, v = acc[:, 0:P], acc[:, P:2 * P], acc[:, 2 * P:3 * P]
            q_s[:, rows] = rope_t(q.T, sb)
            k_s[rows, :] = rope_t(k.T, sb).T
            v_s[:, rows] = v.T

        for sb in range(nsb):
            rows = pl.ds(sb * SB, SB)
            sT = jax.lax.dot_general(
                k_s[...], q_s[:, rows], (((1,), (0,)), ((), ())),
                preferred_element_type=jnp.float32)
            p = jnp.exp(sT * scale)
            denom = jnp.sum(p, axis=0, keepdims=True)
            oT = jax.lax.dot_general(
                v_s[...], p.astype(jnp.bfloat16), (((1,), (0,)), ((), ())),
                preferred_element_type=jnp.float32)
            oh = (oT * (1.0 / denom)).T.astype(jnp.bfloat16)
            wh = woutT_ref[:, rows].T.astype(jnp.bfloat16)
            ow = jnp.concatenate(
                [oh * wh[:, e:e + 1] for e in range(E)], axis=1)
            contrib = jax.lax.dot_general(
                ow, wos_s[...], (((1,), (0,)), ((), ())),
                preferred_element_type=jnp.float32)

            @pl.when(h == 0)
            def _first():
                acc_s[rows, :] = contrib

            @pl.when(h > 0)
            def _rest():
                acc_s[rows, :] = acc_s[rows, :] + contrib

    def _ln1():
        for sb in range(nsb):
            rows = pl.ds(sb * SB, SB)
            x1 = x_ref[rows, :] + acc_s[rows, :]
            mu = jnp.mean(x1, axis=-1, keepdims=True)
            xc = x1 - mu
            var = jnp.mean(xc * xc, axis=-1, keepdims=True)
            xn = xc * jax.lax.rsqrt(var + 1e-5) * g1_ref[...] + bb1_ref[...]
            xn_s[rows, :] = xn.astype(jnp.bfloat16)
        for sb in range(nsb):
            rows = pl.ds(sb * SB, SB)
            acc_s[rows, :] = jnp.zeros((SB, D), jnp.float32) + b2_ref[...]

    def _do_ffn(last):
        w1c = w1_ref[...].astype(jnp.bfloat16)
        w2c = w2_ref[...].astype(jnp.bfloat16)
        for sb in range(nsb):
            rows = pl.ds(sb * SB, SB)
            h1 = jax.lax.dot_general(
                xn_s[rows, :], w1c, (((1,), (0,)), ((), ())),
                preferred_element_type=jnp.float32) + b1_ref[...]
            h1 = jnp.maximum(h1, 0.0).astype(jnp.bfloat16)
            acc_s[rows, :] = acc_s[rows, :] + jax.lax.dot_general(
                h1, w2c, (((1,), (0,)), ((), ())),
                preferred_element_type=jnp.float32)
        if last:
            for sb in range(nsb):
                rows = pl.ds(sb * SB, SB)
                x2 = xn_s[rows, :].astype(jnp.float32) + acc_s[rows, :]
                mu2 = jnp.mean(x2, axis=-1, keepdims=True)
                xc2 = x2 - mu2
                var2 = jnp.mean(xc2 * xc2, axis=-1, keepdims=True)
                out_ref[rows, :] = (xc2 * jax.lax.rsqrt(var2 + 1e-5)
                                    * g2_ref[...] + bb2_ref[...])

    @pl.when(h < nheads)
    def _phase_head():
        _do_head()

    @pl.when(h == nheads - 1)
    def _phase_ln1():
        _ln1()

    @pl.when((h >= nheads) & (h < nh - 1))
    def _phase_ffn():
        _do_ffn(False)

    @pl.when(h == nh - 1)
    def _phase_ffn_last():
        _do_ffn(True)


def kernel(src, Wq, Wk, Wv, Wo, sel_w, sel_o_w, W1, b1, W2, b2,
           ln1_g, ln1_b, ln2_g, ln2_b):
    Bb, S, D = src.shape
    H, E, _, P = Wq.shape
    FF = W1.shape[1]
    SB = min(512, S)
    nsb = S // SB
    x = src.reshape(S, D)

    selcat = jnp.concatenate([sel_w, sel_o_w], axis=1)
    pos = jnp.arange(S, dtype=jnp.float32)
    inv = BASE ** (-jnp.arange(HALF, dtype=jnp.float32) / HALF)
    ang = inv[:, None] * pos[None, :]
    cosT_t = jnp.cos(ang).astype(jnp.bfloat16)
    sinT_t = jnp.sin(ang).astype(jnp.bfloat16)
    b1r = b1.reshape(1, FF)
    b2r = b2.reshape(1, D)
    g1 = ln1_g.reshape(1, D)
    bb1 = ln1_b.reshape(1, D)
    g2 = ln2_g.reshape(1, D)
    bb2 = ln2_b.reshape(1, D)

    logits = pl.pallas_call(
        _router_body,
        grid=(1,),
        in_specs=[
            pl.BlockSpec((S, D), lambda i: (0, 0)),
            pl.BlockSpec((D, 2 * H * E), lambda i: (0, 0)),
        ],
        out_specs=pl.BlockSpec((S, 2 * H * E), lambda i: (0, 0)),
        out_shape=jax.ShapeDtypeStruct((S, 2 * H * E), jnp.float32),
    )(x, selcat)
    wselT = _make_sc_top2(S, 2 * H * E)(logits.T)

    NF = 8
    FFC = FF // NF
    import functools as _ft
    out = pl.pallas_call(
        _ft.partial(_mega_body, nheads=H),
        grid=(H + NF,),
        in_specs=[
            pl.BlockSpec((S, D), lambda h: (0, 0)),
            pl.BlockSpec((HALF, S), lambda h: (0, 0)),
            pl.BlockSpec((HALF, S), lambda h: (0, 0)),
            pl.BlockSpec((1, E, D, P), lambda h: (jnp.minimum(h, 11), 0, 0, 0)),
            pl.BlockSpec((1, E, D, P), lambda h: (jnp.minimum(h, 11), 0, 0, 0)),
            pl.BlockSpec((1, E, D, P), lambda h: (jnp.minimum(h, 11), 0, 0, 0)),
            pl.BlockSpec((1, E, P, D), lambda h: (jnp.minimum(h, 11), 0, 0, 0)),
            pl.BlockSpec((E, S), lambda h: (jnp.minimum(h, 11), 0)),
            pl.BlockSpec((E, S), lambda h: (12 + jnp.minimum(h, 11), 0)),
            pl.BlockSpec((1, D), lambda h: (0, 0)),
            pl.BlockSpec((1, D), lambda h: (0, 0)),
            pl.BlockSpec((D, FFC), lambda h: (0, jnp.clip(h - 12, 0, 7))),
            pl.BlockSpec((1, FFC), lambda h: (0, jnp.clip(h - 12, 0, 7))),
            pl.BlockSpec((FFC, D), lambda h: (jnp.clip(h - 12, 0, 7), 0)),
            pl.BlockSpec((1, D), lambda h: (0, 0)),
            pl.BlockSpec((1, D), lambda h: (0, 0)),
            pl.BlockSpec((1, D), lambda h: (0, 0)),
        ],
        out_specs=pl.BlockSpec((S, D), lambda h: (0, 0)),
        out_shape=jax.ShapeDtypeStruct((S, D), jnp.float32),
        scratch_shapes=[
            pltpu.VMEM((S, D), jnp.bfloat16),
            pltpu.VMEM((D, E * 3 * P), jnp.bfloat16),
            pltpu.VMEM((E * P, D), jnp.bfloat16),
            pltpu.VMEM((P, S), jnp.bfloat16),
            pltpu.VMEM((S, P), jnp.bfloat16),
            pltpu.VMEM((P, S), jnp.bfloat16),
            pltpu.VMEM((S, D), jnp.float32),
            pltpu.VMEM((S, D), jnp.bfloat16),
        ],
        compiler_params=pltpu.CompilerParams(
            dimension_semantics=("arbitrary",)),
    )(x, cosT_t, sinT_t, Wq, Wk, Wv, Wo, wselT, wselT, g1, bb1,
      W1, b1r, W2, b2r, g2, bb2)

    return out.reshape(Bb, S, D)


# speedup vs baseline: 1.1469x; 1.1469x over previous
import jax
import jax.numpy as jnp
from jax.experimental import pallas as pl
from jax.experimental.pallas import tpu as pltpu
from jax.experimental.pallas import tpu_sc as plsc
import functools
from jax import lax

ROT = 32
HALF = ROT // 2
BASE = 10000.0


def _router_body(x_ref, selcat_ref, logits_ref):
    logits_ref[...] = jax.lax.dot_general(
        x_ref[...].astype(jnp.bfloat16), selcat_ref[...].astype(jnp.bfloat16),
        (((1,), (0,)), ((), ())),
        preferred_element_type=jnp.float32)


def _make_sc_top2(S, Z):
    info = plsc.get_sparse_core_info()
    NC, NS = info.num_cores, info.num_subcores
    NW = NC * NS
    CB = S // (NW // 2)
    ZH = Z // 2
    mesh = plsc.VectorSubcoreMesh(core_axis_name="c", subcore_axis_name="s")

    @functools.partial(
        pl.kernel,
        out_type=jax.ShapeDtypeStruct((Z, S), jnp.float32),
        mesh=mesh,
        scratch_types=[
            pltpu.VMEM((ZH, CB), jnp.float32),
            pltpu.VMEM((ZH, CB), jnp.float32),
        ],
    )
    def sc_top2(logitsT_hbm, out_hbm, lv, ov):
        wid = lax.axis_index("s") * NC + lax.axis_index("c")
        p = wid // 2
        q = wid % 2
        t0 = p * CB
        r0 = q * ZH
        pltpu.sync_copy(logitsT_hbm.at[pl.ds(r0, ZH), pl.ds(t0, CB)], lv)
        for g in range(ZH // 8):
            for tb in range(CB // 16):
                cols = pl.ds(tb * 16, 16)
                ls = [lv[g * 8 + e, cols] for e in range(8)]
                m1 = ls[0]
                a1 = jnp.zeros((16,), jnp.int32)
                m2 = jnp.full((16,), -1e30, jnp.float32)
                a2 = jnp.full((16,), 8, jnp.int32)
                for e in range(1, 8):
                    gt1 = ls[e] > m1
                    gt2 = ls[e] > m2
                    a2 = jnp.where(gt1, a1, jnp.where(gt2, e, a2))
                    m2 = jnp.where(gt1, m1, jnp.where(gt2, ls[e], m2))
                    a1 = jnp.where(gt1, e, a1)
                    m1 = jnp.where(gt1, ls[e], m1)
                for e in range(8):
                    sel = (a1 == e) | (a2 == e)
                    ov[g * 8 + e, cols] = jnp.where(
                        sel, 1.0 / (1.0 + jnp.exp(-ls[e])), 0.0)
        pltpu.sync_copy(ov, out_hbm.at[pl.ds(r0, ZH), pl.ds(t0, CB)])

    return sc_top2


def _mega_body(x_ref, cosT_ref, sinT_ref, wq_ref, wk_ref, wv_ref, wo_ref,
               wselT_ref, woutT_ref, g1_ref, bb1_ref,
               w1_ref, b1_ref, w2_ref, b2_ref, g2_ref, bb2_ref, out_ref,
               xh_s, wcat_s, wos_s, q_s, k_s, v_s, acc_s, xn_s, *, nheads):
    h = pl.program_id(0)
    nh = pl.num_programs(0)
    S, D = x_ref.shape
    E = wq_ref.shape[1]
    P = wq_ref.shape[3]
    G = 3 * P
    SB = min(512, S)
    nsb = S // SB
    scale = P ** -0.5

    @pl.when(h == 0)
    def _init():
        xh_s[...] = x_ref[...].astype(jnp.bfloat16)

    def _do_head():
        for e in range(E):
            wcat_s[:, e * G:e * G + P] = wq_ref[0, e].astype(jnp.bfloat16)
            wcat_s[:, e * G + P:e * G + 2 * P] = wk_ref[0, e].astype(jnp.bfloat16)
            wcat_s[:, e * G + 2 * P:(e + 1) * G] = wv_ref[0, e].astype(jnp.bfloat16)
        wos_s[...] = jnp.concatenate(
            [wo_ref[0, e] for e in range(E)], axis=0).astype(jnp.bfloat16)

        cosT = cosT_ref[...]
        sinT = sinT_ref[...]

        def rope_t(tt, sb):
            c = cosT[:, sb * SB:(sb + 1) * SB]
            s = sinT[:, sb * SB:(sb + 1) * SB]
            t1 = tt[0:HALF, :]
            t2 = tt[HALF:ROT, :]
            return jnp.concatenate(
                [t1 * c - t2 * s, t1 * s + t2 * c, tt[ROT:, :]], axis=0)

        for sb in range(nsb):
            rows = pl.ds(sb * SB, SB)
            xb = xh_s[rows, :]
            qkv = jax.lax.dot_general(
                xb, wcat_s[...], (((1,), (0,)), ((), ())),
                preferred_element_type=jnp.float32).astype(jnp.bfloat16)
            w = wselT_ref[:, rows].T.astype(jnp.bfloat16)
            acc = qkv[:, 0:G] * w[:, 0:1]
            for e in range(1, E):
                acc = acc + qkv[:, e * G:(e + 1) * G] * w[:, e:e + 1]
            q, k, v = acc[:, 0:P], acc[:, P:2 * P], acc[:, 2 * P:3 * P]
            q_s[:, rows] = rope_t(q.T, sb) * jnp.bfloat16(scale)
            k_s[rows, :] = rope_t(k.T, sb).T
            v_s[:, rows] = v.T

        for sb in range(nsb):
            rows = pl.ds(sb * SB, SB)
            sT = jax.lax.dot_general(
                k_s[...], q_s[:, rows], (((1,), (0,)), ((), ())),
                preferred_element_type=jnp.float32)
            p = jnp.exp(sT)
            denom = jnp.sum(p, axis=0, keepdims=True)
            oT = jax.lax.dot_general(
                v_s[...], p.astype(jnp.bfloat16), (((1,), (0,)), ((), ())),
                preferred_element_type=jnp.float32)
            oh = (oT * (1.0 / denom)).T.astype(jnp.bfloat16)
            wh = woutT_ref[:, rows].T.astype(jnp.bfloat16)
            ow = jnp.concatenate(
                [oh * wh[:, e:e + 1] for e in range(E)], axis=1)
            contrib = jax.lax.dot_general(
                ow, wos_s[...], (((1,), (0,)), ((), ())),
                preferred_element_type=jnp.float32)

            @pl.when(h == 0)
            def _first():
                acc_s[rows, :] = contrib

            @pl.when(h > 0)
            def _rest():
                acc_s[rows, :] = acc_s[rows, :] + contrib

    def _ln1():
        for sb in range(nsb):
            rows = pl.ds(sb * SB, SB)
            x1 = x_ref[rows, :] + acc_s[rows, :]
            mu = jnp.mean(x1, axis=-1, keepdims=True)
            xc = x1 - mu
            var = jnp.mean(xc * xc, axis=-1, keepdims=True)
            xn = xc * jax.lax.rsqrt(var + 1e-5) * g1_ref[...] + bb1_ref[...]
            xn_s[rows, :] = xn.astype(jnp.bfloat16)
        for sb in range(nsb):
            rows = pl.ds(sb * SB, SB)
            acc_s[rows, :] = jnp.zeros((SB, D), jnp.float32) + b2_ref[...]

    def _do_ffn(last):
        w1c = w1_ref[...].astype(jnp.bfloat16)
        w2c = w2_ref[...].astype(jnp.bfloat16)
        for sb in range(nsb):
            rows = pl.ds(sb * SB, SB)
            h1 = jax.lax.dot_general(
                xn_s[rows, :], w1c, (((1,), (0,)), ((), ())),
                preferred_element_type=jnp.float32) + b1_ref[...]
            h1 = jnp.maximum(h1, 0.0).astype(jnp.bfloat16)
            acc_s[rows, :] = acc_s[rows, :] + jax.lax.dot_general(
                h1, w2c, (((1,), (0,)), ((), ())),
                preferred_element_type=jnp.float32)
        if last:
            for sb in range(nsb):
                rows = pl.ds(sb * SB, SB)
                x2 = xn_s[rows, :].astype(jnp.float32) + acc_s[rows, :]
                mu2 = jnp.mean(x2, axis=-1, keepdims=True)
                xc2 = x2 - mu2
                var2 = jnp.mean(xc2 * xc2, axis=-1, keepdims=True)
                out_ref[rows, :] = (xc2 * jax.lax.rsqrt(var2 + 1e-5)
                                    * g2_ref[...] + bb2_ref[...])

    @pl.when(h < nheads)
    def _phase_head():
        _do_head()

    @pl.when(h == nheads - 1)
    def _phase_ln1():
        _ln1()

    @pl.when((h >= nheads) & (h < nh - 1))
    def _phase_ffn():
        _do_ffn(False)

    @pl.when(h == nh - 1)
    def _phase_ffn_last():
        _do_ffn(True)


def kernel(src, Wq, Wk, Wv, Wo, sel_w, sel_o_w, W1, b1, W2, b2,
           ln1_g, ln1_b, ln2_g, ln2_b):
    Bb, S, D = src.shape
    H, E, _, P = Wq.shape
    FF = W1.shape[1]
    SB = min(512, S)
    nsb = S // SB
    x = src.reshape(S, D)

    selcat = jnp.concatenate([sel_w, sel_o_w], axis=1)
    pos = jnp.arange(S, dtype=jnp.float32)
    inv = BASE ** (-jnp.arange(HALF, dtype=jnp.float32) / HALF)
    ang = inv[:, None] * pos[None, :]
    cosT_t = jnp.cos(ang).astype(jnp.bfloat16)
    sinT_t = jnp.sin(ang).astype(jnp.bfloat16)
    b1r = b1.reshape(1, FF)
    b2r = b2.reshape(1, D)
    g1 = ln1_g.reshape(1, D)
    bb1 = ln1_b.reshape(1, D)
    g2 = ln2_g.reshape(1, D)
    bb2 = ln2_b.reshape(1, D)

    logits = pl.pallas_call(
        _router_body,
        grid=(1,),
        in_specs=[
            pl.BlockSpec((S, D), lambda i: (0, 0)),
            pl.BlockSpec((D, 2 * H * E), lambda i: (0, 0)),
        ],
        out_specs=pl.BlockSpec((S, 2 * H * E), lambda i: (0, 0)),
        out_shape=jax.ShapeDtypeStruct((S, 2 * H * E), jnp.float32),
    )(x, selcat)
    wselT = _make_sc_top2(S, 2 * H * E)(logits.T)

    NF = 8
    FFC = FF // NF
    import functools as _ft
    out = pl.pallas_call(
        _ft.partial(_mega_body, nheads=H),
        grid=(H + NF,),
        in_specs=[
            pl.BlockSpec((S, D), lambda h: (0, 0)),
            pl.BlockSpec((HALF, S), lambda h: (0, 0)),
            pl.BlockSpec((HALF, S), lambda h: (0, 0)),
            pl.BlockSpec((1, E, D, P), lambda h: (jnp.minimum(h, 11), 0, 0, 0)),
            pl.BlockSpec((1, E, D, P), lambda h: (jnp.minimum(h, 11), 0, 0, 0)),
            pl.BlockSpec((1, E, D, P), lambda h: (jnp.minimum(h, 11), 0, 0, 0)),
            pl.BlockSpec((1, E, P, D), lambda h: (jnp.minimum(h, 11), 0, 0, 0)),
            pl.BlockSpec((E, S), lambda h: (jnp.minimum(h, 11), 0)),
            pl.BlockSpec((E, S), lambda h: (12 + jnp.minimum(h, 11), 0)),
            pl.BlockSpec((1, D), lambda h: (0, 0)),
            pl.BlockSpec((1, D), lambda h: (0, 0)),
            pl.BlockSpec((D, FFC), lambda h: (0, jnp.clip(h - 12, 0, 7))),
            pl.BlockSpec((1, FFC), lambda h: (0, jnp.clip(h - 12, 0, 7))),
            pl.BlockSpec((FFC, D), lambda h: (jnp.clip(h - 12, 0, 7), 0)),
            pl.BlockSpec((1, D), lambda h: (0, 0)),
            pl.BlockSpec((1, D), lambda h: (0, 0)),
            pl.BlockSpec((1, D), lambda h: (0, 0)),
        ],
        out_specs=pl.BlockSpec((S, D), lambda h: (0, 0)),
        out_shape=jax.ShapeDtypeStruct((S, D), jnp.float32),
        scratch_shapes=[
            pltpu.VMEM((S, D), jnp.bfloat16),
            pltpu.VMEM((D, E * 3 * P), jnp.bfloat16),
            pltpu.VMEM((E * P, D), jnp.bfloat16),
            pltpu.VMEM((P, S), jnp.bfloat16),
            pltpu.VMEM((S, P), jnp.bfloat16),
            pltpu.VMEM((P, S), jnp.bfloat16),
            pltpu.VMEM((S, D), jnp.float32),
            pltpu.VMEM((S, D), jnp.bfloat16),
        ],
        compiler_params=pltpu.CompilerParams(
            dimension_semantics=("arbitrary",)),
    )(x, cosT_t, sinT_t, Wq, Wk, Wv, Wo, wselT, wselT, g1, bb1,
      W1, b1r, W2, b2r, g2, bb2)

    return out.reshape(Bb, S, D)
